# Initial kernel scaffold; baseline (speedup 1.0000x reference)
import jax
import jax.numpy as jnp
from jax.experimental import pallas as pl


def kernel(encoder_features, adj_weight, W_gcn, W_disc, b_disc, edge_index, pseudo_labels):
    raise NotImplementedError("write your pallas kernel here")


# speedup vs baseline: 5.7776x; 5.7776x over previous
import functools

import jax
import jax.numpy as jnp
from jax import lax
from jax.experimental import pallas as pl
from jax.experimental.pallas import tpu as pltpu
from jax.experimental.pallas import tpu_sc as plsc

NC = 2
NS = 16
NW = NC * NS


def _support_body(x_ref, w_ref, out_ref):
    out_ref[...] = jnp.dot(x_ref[...], w_ref[...],
                           preferred_element_type=jnp.float32)


def _support_matmul(x, w):
    n, d = x.shape
    return pl.pallas_call(
        _support_body,
        out_shape=jax.ShapeDtypeStruct((n, d), jnp.float32),
    )(x, w)


def _make_seg_sum(n, d, e):
    epw = e // NW
    assert epw * NW == e
    blk = 80
    nblk = epw // blk
    assert nblk * blk == epw
    rows_pt = (n // NS) // 8 * 8
    rows_rem = n - rows_pt * NS
    assert rows_rem >= 0 and rows_rem % 8 == 0
    mesh = plsc.VectorSubcoreMesh(core_axis_name="c", subcore_axis_name="s")

    @functools.partial(
        pl.kernel,
        out_type=jax.ShapeDtypeStruct((NC, n, d), jnp.float32),
        mesh=mesh,
        scratch_types=[
            pltpu.VMEM_SHARED((n, d), jnp.float32),
            pltpu.VMEM((epw,), jnp.int32),
            pltpu.VMEM((epw,), jnp.float32),
            pltpu.VMEM((blk,), jnp.int32),
            pltpu.VMEM((blk, d), jnp.float32),
            pltpu.SemaphoreType.DMA,
        ],
    )
    def seg(support_hbm, src_hbm, dst_hbm, w_hbm, zeros_hbm, out_hbm,
            acc, src_v, w_v, dst_v, rows_v, sem):
        cid = lax.axis_index("c")
        sid = lax.axis_index("s")
        wid = cid * NS + sid
        base0 = wid * epw

        @pl.when(sid == 0)
        def _():
            pltpu.sync_copy(zeros_hbm, acc)
        plsc.subcore_barrier()

        pltpu.sync_copy(src_hbm.at[pl.ds(base0, epw)], src_v)
        pltpu.sync_copy(w_hbm.at[pl.ds(base0, epw)], w_v)

        def blk_body(b, carry):
            base = b * blk
            pltpu.sync_copy(dst_hbm.at[pl.ds(base0 + base, blk)], dst_v)
            pltpu.async_copy(
                support_hbm.at[src_v.at[pl.ds(base, blk)]], rows_v, sem
            ).wait()

            def grp_body(t, c2):
                wv = w_v[pl.ds(base + t * 16, 16)]
                for jj in range(16):
                    wj = wv[jj]
                    row = t * 16 + jj
                    for k in range(d // 16):
                        sl = pl.ds(k * 16, 16)
                        rows_v[row, sl] = rows_v[row, sl] * wj
                return c2

            lax.fori_loop(0, blk // 16, grp_body, 0)
            pltpu.sync_copy(rows_v, acc.at[dst_v], add=True)
            return carry

        lax.fori_loop(0, nblk, blk_body, 0)
        plsc.subcore_barrier()
        off = pl.multiple_of(sid * rows_pt, 8)
        pltpu.sync_copy(
            acc.at[pl.ds(off, rows_pt)],
            out_hbm.at[cid, pl.ds(off, rows_pt)],
        )
        if rows_rem:
            @pl.when(sid == 0)
            def _():
                pltpu.sync_copy(
                    acc.at[pl.ds(rows_pt * NS, rows_rem)],
                    out_hbm.at[cid, pl.ds(rows_pt * NS, rows_rem)],
                )

    return seg


def _loss_body(p_ref, lbl_ref, w_ref, b_ref, out_ref):
    i = pl.program_id(0)
    e = p_ref[0] + p_ref[1]
    logits = jnp.dot(e, w_ref[...], preferred_element_type=jnp.float32)
    logits = logits + b_ref[...]
    m = jnp.max(logits, axis=1, keepdims=True)
    ex = jnp.exp(logits - m)
    lse = jnp.log(jnp.sum(ex, axis=1, keepdims=True)) + m
    col = lax.broadcasted_iota(jnp.int32, logits.shape, 1)
    oh = col == lbl_ref[...]
    part = jnp.sum(jnp.where(oh, logits, 0.0)) - jnp.sum(lse)

    @pl.when(i == 0)
    def _():
        out_ref[0, 0] = 0.0

    out_ref[0, 0] += part


def _loss(partials, labels2d, w_pad, b_pad):
    _, n, d = partials.shape
    cpad = w_pad.shape[1]
    r = 2000
    grid = n // r
    assert grid * r == n
    return pl.pallas_call(
        _loss_body,
        grid=(grid,),
        in_specs=[
            pl.BlockSpec((NC, r, d), lambda i: (0, i, 0)),
            pl.BlockSpec((r, 1), lambda i: (i, 0)),
            pl.BlockSpec((d, cpad), lambda i: (0, 0)),
            pl.BlockSpec((1, cpad), lambda i: (0, 0)),
        ],
        out_specs=pl.BlockSpec((1, 1), lambda i: (0, 0),
                               memory_space=pltpu.SMEM),
        out_shape=jax.ShapeDtypeStruct((1, 1), jnp.float32),
    )(partials, labels2d, w_pad, b_pad)


def kernel(encoder_features, adj_weight, W_gcn, W_disc, b_disc, edge_index,
           pseudo_labels):
    n, d = encoder_features.shape
    e = edge_index.shape[1]
    nparts = W_disc.shape[1]
    cpad = ((nparts + 127) // 128) * 128

    support = _support_matmul(encoder_features, W_gcn)

    seg = _make_seg_sum(n, d, e)
    zeros = jnp.zeros((n, d), jnp.float32)
    partials = seg(support, edge_index[0], edge_index[1], adj_weight, zeros)

    w_pad = jnp.concatenate(
        [W_disc, jnp.zeros((d, cpad - nparts), jnp.float32)], axis=1)
    b_pad = jnp.concatenate(
        [b_disc, jnp.full((cpad - nparts,), -jnp.inf, jnp.float32)])[None, :]
    labels2d = pseudo_labels.astype(jnp.int32)[:, None]

    acc = _loss(partials, labels2d, w_pad, b_pad)
    return -acc[0, 0] / n


# speedup vs baseline: 10.2616x; 1.7761x over previous
import functools

import jax
import jax.numpy as jnp
from jax import lax
from jax.experimental import pallas as pl
from jax.experimental.pallas import tpu as pltpu
from jax.experimental.pallas import tpu_sc as plsc

NC = 2
NS = 16
NW = NC * NS
BLK = 64
NBUF = 4


def _support_body(x_ref, w_ref, out_ref):
    out_ref[...] = jnp.dot(x_ref[...], w_ref[...],
                           preferred_element_type=jnp.float32)


def _support_matmul(x, w):
    n, d = x.shape
    return pl.pallas_call(
        _support_body,
        out_shape=jax.ShapeDtypeStruct((n, d), jnp.float32),
    )(x, w)


def _make_seg_sum(n, d, e):
    ept = e // NW
    nmain = ept // BLK
    tail = ept - nmain * BLK
    assert ept * NW == e and tail % 8 == 0 and tail <= 32
    main_e = NW * nmain * BLK
    rows_pt = (n // NS) // 8 * 8
    rows_rem = n - rows_pt * NS
    assert rows_rem >= 0 and rows_rem % 8 == 0
    mesh = plsc.VectorSubcoreMesh(core_axis_name="c", subcore_axis_name="s")

    @functools.partial(
        pl.kernel,
        out_type=jax.ShapeDtypeStruct((NC, n, d), jnp.float32),
        mesh=mesh,
        scratch_types=[
            pltpu.VMEM_SHARED((n, d), jnp.float32),
            pltpu.VMEM((ept,), jnp.int32),
            pltpu.VMEM((NBUF, BLK), jnp.float32),
            pltpu.VMEM((32,), jnp.float32),
            pltpu.VMEM((NBUF, BLK), jnp.int32),
            pltpu.VMEM((32,), jnp.int32),
            pltpu.VMEM((NBUF, BLK, d), jnp.float32),
            pltpu.VMEM((32, d), jnp.float32),
            pltpu.SemaphoreType.DMA((NBUF,)),
            pltpu.SemaphoreType.DMA((NBUF,)),
            pltpu.SemaphoreType.DMA((NBUF,)),
            pltpu.SemaphoreType.DMA((NBUF,)),
        ],
    )
    def seg(support_hbm, src_hbm, dst3_hbm, dstt_hbm, w_hbm, zeros_hbm,
            out_hbm, acc, src_v, w_r, wt_v, dst_r, dstt_v, rows, rows_t,
            gsem, ssem, dsem, wsem):
        cid = lax.axis_index("c")
        sid = lax.axis_index("s")
        wid = cid * NS + sid
        base_m = pl.multiple_of(wid * (nmain * BLK), 8)
        base_t = pl.multiple_of(main_e + wid * tail, 8)

        pltpu.sync_copy(src_hbm.at[pl.ds(base_m, nmain * BLK)],
                        src_v.at[pl.ds(0, nmain * BLK)])
        pltpu.sync_copy(src_hbm.at[pl.ds(base_t, tail)],
                        src_v.at[pl.ds(nmain * BLK, tail)])
        pltpu.sync_copy(w_hbm.at[pl.ds(base_t, tail)],
                        wt_v.at[pl.ds(0, tail)])
        pltpu.sync_copy(dstt_hbm.at[pl.ds(wid * tail, tail)],
                        dstt_v.at[pl.ds(0, tail)])

        off = pl.multiple_of(sid * rows_pt, 8)
        pltpu.sync_copy(zeros_hbm.at[pl.ds(off, rows_pt)],
                        acc.at[pl.ds(off, rows_pt)])
        if rows_rem:
            @pl.when(sid == 0)
            def _():
                pltpu.sync_copy(zeros_hbm.at[pl.ds(rows_pt * NS, rows_rem)],
                                acc.at[pl.ds(rows_pt * NS, rows_rem)])
        plsc.subcore_barrier()

        def g_desc(b, j):
            return pltpu.make_async_copy(
                support_hbm.at[src_v.at[pl.ds(b * BLK, BLK)]],
                rows.at[j], gsem.at[j])

        def d_desc(b, j):
            return pltpu.make_async_copy(
                dst3_hbm.at[wid, b], dst_r.at[j], dsem.at[j])

        def w_desc(b, j):
            return pltpu.make_async_copy(
                w_hbm.at[pl.ds(base_m + b * BLK, BLK)], w_r.at[j],
                wsem.at[j])

        def s_desc(b, j):
            return pltpu.make_async_copy(
                rows.at[j], acc.at[dst_r.at[j]], ssem.at[j])

        def mult(buf_ref, w_ref, nrow):
            def grp(t, c2):
                wv = w_ref[pl.ds(t * 16, 16)]
                for jj in range(16):
                    wj = wv[jj]
                    row = t * 16 + jj
                    for k in range(d // 16):
                        sl = pl.ds(k * 16, 16)
                        buf_ref[row, sl] = buf_ref[row, sl] * wj
                return c2

            lax.fori_loop(0, nrow // 16, grp, 0)

        def step(b, j):
            g_desc(b, j).wait()
            d_desc(b, j).wait()
            w_desc(b, j).wait()
            mult(rows.at[j], w_r.at[j], BLK)
            s_desc(b, j).start(add=True)
            jn = (j + 2) % NBUF

            @pl.when(b >= 2)
            def _():
                s_desc(b - 2, jn).wait()

            @pl.when(b + 2 < nmain)
            def _():
                d_desc(b + 2, jn).start()
                w_desc(b + 2, jn).start()
                g_desc(b + 2, jn).start()

        for b in range(2):
            d_desc(b, b).start()
            w_desc(b, b).start()
            g_desc(b, b).start()

        nquad = nmain // NBUF

        def quad(q, c):
            for j in range(NBUF):
                step(q * NBUF + j, j)
            return c

        lax.fori_loop(0, nquad, quad, 0)
        for b in range(nquad * NBUF, nmain):
            step(b, b % NBUF)
        s_desc(nmain - 2, (nmain - 2) % NBUF).wait()
        s_desc(nmain - 1, (nmain - 1) % NBUF).wait()

        if tail:
            tdesc = pltpu.make_async_copy(
                support_hbm.at[src_v.at[pl.ds(nmain * BLK, tail)]],
                rows_t.at[pl.ds(0, tail)], gsem.at[0])
            tdesc.start()
            tdesc.wait()
            for t in range(tail // 16):
                wv = wt_v[pl.ds(t * 16, 16)]
                for jj in range(16):
                    wj = wv[jj]
                    row = t * 16 + jj
                    for k in range(d // 16):
                        sl = pl.ds(k * 16, 16)
                        rows_t[row, sl] = rows_t[row, sl] * wj
            pltpu.sync_copy(rows_t.at[pl.ds(0, tail)],
                            acc.at[dstt_v.at[pl.ds(0, tail)]], add=True)

        plsc.subcore_barrier()
        pltpu.sync_copy(
            acc.at[pl.ds(off, rows_pt)],
            out_hbm.at[cid, pl.ds(off, rows_pt)],
        )
        if rows_rem:
            @pl.when(sid == 0)
            def _():
                pltpu.sync_copy(
                    acc.at[pl.ds(rows_pt * NS, rows_rem)],
                    out_hbm.at[cid, pl.ds(rows_pt * NS, rows_rem)],
                )

    return seg


def _loss_body(p_ref, lbl_ref, w_ref, b_ref, out_ref):
    i = pl.program_id(0)
    emb = p_ref[0] + p_ref[1]
    logits = jnp.dot(emb, w_ref[...], preferred_element_type=jnp.float32)
    logits = logits + b_ref[...]
    m = jnp.max(logits, axis=1, keepdims=True)
    ex = jnp.exp(logits - m)
    lse = jnp.log(jnp.sum(ex, axis=1, keepdims=True)) + m
    col = lax.broadcasted_iota(jnp.int32, logits.shape, 1)
    oh = col == lbl_ref[...]
    part = jnp.sum(jnp.where(oh, logits, 0.0)) - jnp.sum(lse)

    @pl.when(i == 0)
    def _():
        out_ref[0, 0] = 0.0

    out_ref[0, 0] += part


def _loss(partials, labels2d, w_pad, b_pad):
    _, n, d = partials.shape
    cpad = w_pad.shape[1]
    r = 2000
    grid = n // r
    assert grid * r == n
    return pl.pallas_call(
        _loss_body,
        grid=(grid,),
        in_specs=[
            pl.BlockSpec((NC, r, d), lambda i: (0, i, 0)),
            pl.BlockSpec((r, 1), lambda i: (i, 0)),
            pl.BlockSpec((d, cpad), lambda i: (0, 0)),
            pl.BlockSpec((1, cpad), lambda i: (0, 0)),
        ],
        out_specs=pl.BlockSpec((1, 1), lambda i: (0, 0),
                               memory_space=pltpu.SMEM),
        out_shape=jax.ShapeDtypeStruct((1, 1), jnp.float32),
    )(partials, labels2d, w_pad, b_pad)


def kernel(encoder_features, adj_weight, W_gcn, W_disc, b_disc, edge_index,
           pseudo_labels):
    n, d = encoder_features.shape
    e = edge_index.shape[1]
    nparts = W_disc.shape[1]
    cpad = ((nparts + 127) // 128) * 128

    support = _support_matmul(encoder_features, W_gcn)

    seg = _make_seg_sum(n, d, e)
    zeros = jnp.zeros((n, d), jnp.float32)
    nmain = (e // NW) // BLK
    main_e = NW * nmain * BLK
    dst = edge_index[1]
    dst3 = dst[:main_e].reshape(NW, nmain, BLK)
    dst_tail = dst[main_e:]
    partials = seg(support, edge_index[0], dst3, dst_tail, adj_weight, zeros)

    w_pad = jnp.concatenate(
        [W_disc, jnp.zeros((d, cpad - nparts), jnp.float32)], axis=1)
    b_pad = jnp.concatenate(
        [b_disc, jnp.full((cpad - nparts,), -jnp.inf, jnp.float32)])[None, :]
    labels2d = pseudo_labels.astype(jnp.int32)[:, None]

    acc = _loss(partials, labels2d, w_pad, b_pad)
    return -acc[0, 0] / n


# speedup vs baseline: 11.7958x; 1.1495x over previous
import functools

import jax
import jax.numpy as jnp
from jax import lax
from jax.experimental import pallas as pl
from jax.experimental.pallas import tpu as pltpu
from jax.experimental.pallas import tpu_sc as plsc

NC = 2
NS = 16
NW = NC * NS
BLK = 64
NBUF = 4


def _support_body(x_ref, w_ref, out_ref):
    out_ref[...] = jnp.dot(x_ref[...], w_ref[...],
                           preferred_element_type=jnp.float32)


def _support_matmul(x, w):
    n, d = x.shape
    return pl.pallas_call(
        _support_body,
        out_shape=jax.ShapeDtypeStruct((n, d), jnp.float32),
    )(x, w)


def _make_seg_sum(n, d, e):
    ept = e // NW
    nmain = ept // BLK
    tail = ept - nmain * BLK
    assert ept * NW == e and tail % 8 == 0 and tail <= 32
    main_e = NW * nmain * BLK
    rows_pt = (n // NS) // 8 * 8
    rows_rem = n - rows_pt * NS
    assert rows_rem >= 0 and rows_rem % 8 == 0
    mesh = plsc.VectorSubcoreMesh(core_axis_name="c", subcore_axis_name="s")

    @functools.partial(
        pl.kernel,
        out_type=jax.ShapeDtypeStruct((NC, n, d), jnp.float32),
        mesh=mesh,
        scratch_types=[
            pltpu.VMEM_SHARED((n, d), jnp.float32),
            pltpu.VMEM((ept,), jnp.int32),
            pltpu.VMEM((NBUF, BLK), jnp.float32),
            pltpu.VMEM((32,), jnp.float32),
            pltpu.VMEM((NBUF, BLK), jnp.int32),
            pltpu.VMEM((32,), jnp.int32),
            pltpu.VMEM((NBUF, BLK, d), jnp.float32),
            pltpu.VMEM((32, d), jnp.float32),
            pltpu.SemaphoreType.DMA((NBUF,)),
            pltpu.SemaphoreType.DMA((NBUF,)),
            pltpu.SemaphoreType.DMA((NBUF,)),
            pltpu.SemaphoreType.DMA((NBUF,)),
        ],
    )
    def seg(support_hbm, src_hbm, dst_hbm, dstt_hbm, w_hbm, zeros_hbm,
            out_hbm, acc, src_v, w_r, wt_v, dst_r, dstt_v, rows, rows_t,
            gsem, ssem, dsem, wsem):
        cid = lax.axis_index("c")
        sid = lax.axis_index("s")
        wid = cid * NS + sid
        base_m = pl.multiple_of(wid * (nmain * BLK), 8)
        base_t = pl.multiple_of(main_e + wid * tail, 8)

        pltpu.sync_copy(src_hbm.at[pl.ds(base_m, nmain * BLK)],
                        src_v.at[pl.ds(0, nmain * BLK)])
        pltpu.sync_copy(src_hbm.at[pl.ds(base_t, tail)],
                        src_v.at[pl.ds(nmain * BLK, tail)])
        pltpu.sync_copy(w_hbm.at[pl.ds(base_t, tail)],
                        wt_v.at[pl.ds(0, tail)])
        pltpu.sync_copy(dstt_hbm.at[pl.ds(wid * tail, tail)],
                        dstt_v.at[pl.ds(0, tail)])

        off = pl.multiple_of(sid * rows_pt, 8)
        pltpu.sync_copy(zeros_hbm.at[pl.ds(off, rows_pt)],
                        acc.at[pl.ds(off, rows_pt)])
        if rows_rem:
            @pl.when(sid == 0)
            def _():
                pltpu.sync_copy(zeros_hbm.at[pl.ds(rows_pt * NS, rows_rem)],
                                acc.at[pl.ds(rows_pt * NS, rows_rem)])
        plsc.subcore_barrier()

        def g_desc(b, j):
            return pltpu.make_async_copy(
                support_hbm.at[src_v.at[pl.ds(b * BLK, BLK)]],
                rows.at[j], gsem.at[j])

        def d_desc(b, j):
            return pltpu.make_async_copy(
                dst_hbm.at[pl.ds(base_m + b * BLK, BLK)], dst_r.at[j],
                dsem.at[j])

        def w_desc(b, j):
            return pltpu.make_async_copy(
                w_hbm.at[pl.ds(base_m + b * BLK, BLK)], w_r.at[j],
                wsem.at[j])

        def s_desc(b, j):
            return pltpu.make_async_copy(
                rows.at[j], acc.at[dst_r.at[j]], ssem.at[j])

        def mult(buf_ref, w_ref, nrow):
            def grp(t, c2):
                wv = w_ref[pl.ds(t * 16, 16)]
                for jj in range(16):
                    wj = wv[jj]
                    row = t * 16 + jj
                    for k in range(d // 16):
                        sl = pl.ds(k * 16, 16)
                        buf_ref[row, sl] = buf_ref[row, sl] * wj
                return c2

            lax.fori_loop(0, nrow // 16, grp, 0)

        def step(b, j):
            g_desc(b, j).wait()
            d_desc(b, j).wait()
            w_desc(b, j).wait()
            jn = (j + 2) % NBUF

            @pl.when(b >= 2)
            def _():
                s_desc(b - 2, jn).wait()

            @pl.when(b + 2 < nmain)
            def _():
                d_desc(b + 2, jn).start()
                w_desc(b + 2, jn).start()
                g_desc(b + 2, jn).start()

            mult(rows.at[j], w_r.at[j], BLK)
            s_desc(b, j).start(add=True)

        for b in range(2):
            d_desc(b, b).start()
            w_desc(b, b).start()
            g_desc(b, b).start()

        nquad = nmain // NBUF

        def quad(q, c):
            for j in range(NBUF):
                step(q * NBUF + j, j)
            return c

        lax.fori_loop(0, nquad, quad, 0)
        for b in range(nquad * NBUF, nmain):
            step(b, b % NBUF)
        s_desc(nmain - 2, (nmain - 2) % NBUF).wait()
        s_desc(nmain - 1, (nmain - 1) % NBUF).wait()

        if tail:
            tdesc = pltpu.make_async_copy(
                support_hbm.at[src_v.at[pl.ds(nmain * BLK, tail)]],
                rows_t.at[pl.ds(0, tail)], gsem.at[0])
            tdesc.start()
            tdesc.wait()
            for t in range(tail // 16):
                wv = wt_v[pl.ds(t * 16, 16)]
                for jj in range(16):
                    wj = wv[jj]
                    row = t * 16 + jj
                    for k in range(d // 16):
                        sl = pl.ds(k * 16, 16)
                        rows_t[row, sl] = rows_t[row, sl] * wj
            pltpu.sync_copy(rows_t.at[pl.ds(0, tail)],
                            acc.at[dstt_v.at[pl.ds(0, tail)]], add=True)

        plsc.subcore_barrier()
        pltpu.sync_copy(
            acc.at[pl.ds(off, rows_pt)],
            out_hbm.at[cid, pl.ds(off, rows_pt)],
        )
        if rows_rem:
            @pl.when(sid == 0)
            def _():
                pltpu.sync_copy(
                    acc.at[pl.ds(rows_pt * NS, rows_rem)],
                    out_hbm.at[cid, pl.ds(rows_pt * NS, rows_rem)],
                )

    return seg


def _loss_body(p_ref, lbl_ref, w_ref, b_ref, out_ref):
    i = pl.program_id(0)
    emb = p_ref[0] + p_ref[1]
    logits = jnp.dot(emb, w_ref[...], preferred_element_type=jnp.float32)
    logits = logits + b_ref[...]
    m = jnp.max(logits, axis=1, keepdims=True)
    ex = jnp.exp(logits - m)
    lse = jnp.log(jnp.sum(ex, axis=1, keepdims=True)) + m
    col = lax.broadcasted_iota(jnp.int32, logits.shape, 1)
    oh = col == lbl_ref[...]
    part = jnp.sum(jnp.where(oh, logits, 0.0)) - jnp.sum(lse)

    @pl.when(i == 0)
    def _():
        out_ref[0, 0] = 0.0

    out_ref[0, 0] += part


def _loss(partials, labels2d, w_pad, b_pad):
    _, n, d = partials.shape
    cpad = w_pad.shape[1]
    r = 2000
    grid = n // r
    assert grid * r == n
    return pl.pallas_call(
        _loss_body,
        grid=(grid,),
        in_specs=[
            pl.BlockSpec((NC, r, d), lambda i: (0, i, 0)),
            pl.BlockSpec((r, 1), lambda i: (i, 0)),
            pl.BlockSpec((d, cpad), lambda i: (0, 0)),
            pl.BlockSpec((1, cpad), lambda i: (0, 0)),
        ],
        out_specs=pl.BlockSpec((1, 1), lambda i: (0, 0),
                               memory_space=pltpu.SMEM),
        out_shape=jax.ShapeDtypeStruct((1, 1), jnp.float32),
    )(partials, labels2d, w_pad, b_pad)


def kernel(encoder_features, adj_weight, W_gcn, W_disc, b_disc, edge_index,
           pseudo_labels):
    n, d = encoder_features.shape
    e = edge_index.shape[1]
    nparts = W_disc.shape[1]
    cpad = ((nparts + 127) // 128) * 128

    support = _support_matmul(encoder_features, W_gcn)

    seg = _make_seg_sum(n, d, e)
    zeros = jnp.zeros((n, d), jnp.float32)
    nmain = (e // NW) // BLK
    main_e = NW * nmain * BLK
    dst = edge_index[1]
    dst_tail = dst[main_e:]
    partials = seg(support, edge_index[0], dst, dst_tail, adj_weight, zeros)

    w_pad = jnp.concatenate(
        [W_disc, jnp.zeros((d, cpad - nparts), jnp.float32)], axis=1)
    b_pad = jnp.concatenate(
        [b_disc, jnp.full((cpad - nparts,), -jnp.inf, jnp.float32)])[None, :]
    labels2d = pseudo_labels.astype(jnp.int32)[:, None]

    acc = _loss(partials, labels2d, w_pad, b_pad)
    return -acc[0, 0] / n


# speedup vs baseline: 11.9916x; 1.0166x over previous
import functools

import jax
import jax.numpy as jnp
from jax import lax
from jax.experimental import pallas as pl
from jax.experimental.pallas import tpu as pltpu
from jax.experimental.pallas import tpu_sc as plsc

NC = 2
NS = 16
NW = NC * NS
BLK = 80
NBUF = 4


def _support_body(x_ref, w_ref, out_ref):
    out_ref[...] = jnp.dot(x_ref[...], w_ref[...],
                           preferred_element_type=jnp.float32)


def _support_matmul(x, w):
    n, d = x.shape
    return pl.pallas_call(
        _support_body,
        out_shape=jax.ShapeDtypeStruct((n, d), jnp.float32),
    )(x, w)


def _make_seg_sum(n, d, e):
    ept = e // NW
    nmain = ept // BLK
    assert ept * NW == e and nmain * BLK == ept and BLK % 8 == 0
    rows_pt = (n // NS) // 8 * 8
    rows_rem = n - rows_pt * NS
    assert rows_rem >= 0 and rows_rem % 8 == 0
    mesh = plsc.VectorSubcoreMesh(core_axis_name="c", subcore_axis_name="s")

    @functools.partial(
        pl.kernel,
        out_type=jax.ShapeDtypeStruct((NC, n, d), jnp.float32),
        mesh=mesh,
        scratch_types=[
            pltpu.VMEM_SHARED((n, d), jnp.float32),
            pltpu.VMEM((NBUF, BLK), jnp.int32),
            pltpu.VMEM((NBUF, BLK), jnp.float32),
            pltpu.VMEM((NBUF, BLK), jnp.int32),
            pltpu.VMEM((NBUF, BLK, d), jnp.float32),
            pltpu.SemaphoreType.DMA((NBUF,)),
            pltpu.SemaphoreType.DMA((NBUF,)),
            pltpu.SemaphoreType.DMA((NBUF,)),
            pltpu.SemaphoreType.DMA((NBUF,)),
            pltpu.SemaphoreType.DMA((NBUF,)),
        ],
    )
    def seg(support_hbm, src_hbm, dst_hbm, w_hbm, zeros_hbm,
            out_hbm, acc, src_r, w_r, dst_r, rows,
            gsem, ssem, dsem, wsem, xsem):
        cid = lax.axis_index("c")
        sid = lax.axis_index("s")
        wid = cid * NS + sid
        base_m = pl.multiple_of(wid * (nmain * BLK), 8)

        off = pl.multiple_of(sid * rows_pt, 8)
        pltpu.sync_copy(zeros_hbm.at[pl.ds(off, rows_pt)],
                        acc.at[pl.ds(off, rows_pt)])
        if rows_rem:
            @pl.when(sid == 0)
            def _():
                pltpu.sync_copy(zeros_hbm.at[pl.ds(rows_pt * NS, rows_rem)],
                                acc.at[pl.ds(rows_pt * NS, rows_rem)])

        def x_desc(b, j):
            return pltpu.make_async_copy(
                src_hbm.at[pl.ds(base_m + b * BLK, BLK)], src_r.at[j],
                xsem.at[j])

        def g_desc(b, j):
            return pltpu.make_async_copy(
                support_hbm.at[src_r.at[j]], rows.at[j], gsem.at[j])

        def d_desc(b, j):
            return pltpu.make_async_copy(
                dst_hbm.at[pl.ds(base_m + b * BLK, BLK)], dst_r.at[j],
                dsem.at[j])

        def w_desc(b, j):
            return pltpu.make_async_copy(
                w_hbm.at[pl.ds(base_m + b * BLK, BLK)], w_r.at[j],
                wsem.at[j])

        def s_desc(b, j):
            return pltpu.make_async_copy(
                rows.at[j], acc.at[dst_r.at[j]], ssem.at[j])

        def mult(buf_ref, w_ref, nrow):
            def grp(t, c2):
                wv = w_ref[pl.ds(t * 16, 16)]
                for jj in range(16):
                    wj = wv[jj]
                    row = t * 16 + jj
                    for k in range(d // 16):
                        sl = pl.ds(k * 16, 16)
                        buf_ref[row, sl] = buf_ref[row, sl] * wj
                return c2

            lax.fori_loop(0, nrow // 16, grp, 0)

        for k in range(NBUF):
            x_desc(k, k).start()
        plsc.subcore_barrier()
        for b in range(2):
            x_desc(b, b).wait()
            d_desc(b, b).start()
            w_desc(b, b).start()
            g_desc(b, b).start()

        def step(b, j):
            g_desc(b, j).wait()
            d_desc(b, j).wait()
            w_desc(b, j).wait()

            @pl.when(b + NBUF < nmain)
            def _():
                x_desc(b + NBUF, j).start()

            jn = (j + 2) % NBUF

            @pl.when(b >= 2)
            def _():
                s_desc(b - 2, jn).wait()

            @pl.when(b + 2 < nmain)
            def _():
                x_desc(b + 2, jn).wait()
                d_desc(b + 2, jn).start()
                w_desc(b + 2, jn).start()
                g_desc(b + 2, jn).start()

            mult(rows.at[j], w_r.at[j], BLK)
            s_desc(b, j).start(add=True)

        nquad = nmain // NBUF

        def quad(q, c):
            for j in range(NBUF):
                step(q * NBUF + j, j)
            return c

        lax.fori_loop(0, nquad, quad, 0)
        for b in range(nquad * NBUF, nmain):
            step(b, b % NBUF)
        s_desc(nmain - 2, (nmain - 2) % NBUF).wait()
        s_desc(nmain - 1, (nmain - 1) % NBUF).wait()

        plsc.subcore_barrier()
        pltpu.sync_copy(
            acc.at[pl.ds(off, rows_pt)],
            out_hbm.at[cid, pl.ds(off, rows_pt)],
        )
        if rows_rem:
            @pl.when(sid == 0)
            def _():
                pltpu.sync_copy(
                    acc.at[pl.ds(rows_pt * NS, rows_rem)],
                    out_hbm.at[cid, pl.ds(rows_pt * NS, rows_rem)],
                )

    return seg


def _loss_body(p_ref, lbl_ref, w_ref, b_ref, out_ref):
    i = pl.program_id(0)
    emb = p_ref[0] + p_ref[1]
    logits = jnp.dot(emb, w_ref[...], preferred_element_type=jnp.float32)
    logits = logits + b_ref[...]
    m = jnp.max(logits, axis=1, keepdims=True)
    ex = jnp.exp(logits - m)
    lse = jnp.log(jnp.sum(ex, axis=1, keepdims=True)) + m
    col = lax.broadcasted_iota(jnp.int32, logits.shape, 1)
    oh = col == lbl_ref[...]
    part = jnp.sum(jnp.where(oh, logits, 0.0)) - jnp.sum(lse)

    @pl.when(i == 0)
    def _():
        out_ref[0, 0] = 0.0

    out_ref[0, 0] += part


def _loss(partials, labels2d, w_pad, b_pad):
    _, n, d = partials.shape
    cpad = w_pad.shape[1]
    r = 2000
    grid = n // r
    assert grid * r == n
    return pl.pallas_call(
        _loss_body,
        grid=(grid,),
        in_specs=[
            pl.BlockSpec((NC, r, d), lambda i: (0, i, 0)),
            pl.BlockSpec((r, 1), lambda i: (i, 0)),
            pl.BlockSpec((d, cpad), lambda i: (0, 0)),
            pl.BlockSpec((1, cpad), lambda i: (0, 0)),
        ],
        out_specs=pl.BlockSpec((1, 1), lambda i: (0, 0),
                               memory_space=pltpu.SMEM),
        out_shape=jax.ShapeDtypeStruct((1, 1), jnp.float32),
    )(partials, labels2d, w_pad, b_pad)


def kernel(encoder_features, adj_weight, W_gcn, W_disc, b_disc, edge_index,
           pseudo_labels):
    n, d = encoder_features.shape
    e = edge_index.shape[1]
    nparts = W_disc.shape[1]
    cpad = ((nparts + 127) // 128) * 128

    support = _support_matmul(encoder_features, W_gcn)

    seg = _make_seg_sum(n, d, e)
    zeros = jnp.zeros((n, d), jnp.float32)
    partials = seg(support, edge_index[0], edge_index[1], adj_weight, zeros)

    w_pad = jnp.concatenate(
        [W_disc, jnp.zeros((d, cpad - nparts), jnp.float32)], axis=1)
    b_pad = jnp.concatenate(
        [b_disc, jnp.full((cpad - nparts,), -jnp.inf, jnp.float32)])[None, :]
    labels2d = pseudo_labels.astype(jnp.int32)[:, None]

    acc = _loss(partials, labels2d, w_pad, b_pad)
    return -acc[0, 0] / n


# speedup vs baseline: 12.4370x; 1.0371x over previous
import functools

import jax
import jax.numpy as jnp
from jax import lax
from jax.experimental import pallas as pl
from jax.experimental.pallas import tpu as pltpu
from jax.experimental.pallas import tpu_sc as plsc

NC = 2
NS = 16
NW = NC * NS
BLK = 80
NBUF = 4


def _support_body(x_ref, w_ref, out_ref):
    out_ref[...] = jnp.dot(x_ref[...], w_ref[...],
                           preferred_element_type=jnp.float32)


def _support_matmul(x, w):
    n, d = x.shape
    return pl.pallas_call(
        _support_body,
        out_shape=jax.ShapeDtypeStruct((n, d), jnp.float32),
    )(x, w)


def _make_seg_sum(n, d, e):
    ept = e // NW
    nmain = ept // BLK
    assert ept * NW == e and nmain * BLK == ept and BLK % 8 == 0
    rows_pt = (n // NS) // 8 * 8
    rows_rem = n - rows_pt * NS
    assert rows_rem >= 0 and rows_rem % 8 == 0
    mesh = plsc.VectorSubcoreMesh(core_axis_name="c", subcore_axis_name="s")

    @functools.partial(
        pl.kernel,
        out_type=jax.ShapeDtypeStruct((NC, n, d), jnp.float32),
        mesh=mesh,
        scratch_types=[
            pltpu.VMEM_SHARED((n, d), jnp.float32),
            pltpu.VMEM((NBUF, BLK), jnp.int32),
            pltpu.VMEM((NBUF, BLK), jnp.float32),
            pltpu.VMEM((NBUF, BLK), jnp.int32),
            pltpu.VMEM((NBUF, BLK, d), jnp.float32),
            pltpu.SemaphoreType.DMA((NBUF,)),
            pltpu.SemaphoreType.DMA((NBUF,)),
            pltpu.SemaphoreType.DMA((NBUF,)),
            pltpu.SemaphoreType.DMA((NBUF,)),
            pltpu.SemaphoreType.DMA((NBUF,)),
        ],
    )
    def seg(support_hbm, src_hbm, dst_hbm, w_hbm,
            out_hbm, acc, src_r, w_r, dst_r, rows,
            gsem, ssem, dsem, wsem, xsem):
        cid = lax.axis_index("c")
        sid = lax.axis_index("s")
        wid = cid * NS + sid
        base_m = pl.multiple_of(wid * (nmain * BLK), 8)

        zero16 = jnp.zeros((16,), jnp.float32)

        def zrow(r, c):
            for k in range(d // 16):
                rows[0, r, pl.ds(k * 16, 16)] = zero16
            return c

        lax.fori_loop(0, BLK, zrow, 0)
        off = pl.multiple_of(sid * rows_pt, 8)
        for z in range(rows_pt // BLK):
            pltpu.sync_copy(rows.at[0],
                            acc.at[pl.ds(off + z * BLK, BLK)])
        zrem = rows_pt - (rows_pt // BLK) * BLK
        if zrem:
            pltpu.sync_copy(rows.at[0, pl.ds(0, zrem)],
                            acc.at[pl.ds(off + (rows_pt // BLK) * BLK, zrem)])
        if rows_rem:
            @pl.when(sid == 0)
            def _():
                pltpu.sync_copy(rows.at[0, pl.ds(0, rows_rem)],
                                acc.at[pl.ds(rows_pt * NS, rows_rem)])

        def x_desc(b, j):
            return pltpu.make_async_copy(
                src_hbm.at[pl.ds(base_m + b * BLK, BLK)], src_r.at[j],
                xsem.at[j])

        def g_desc(b, j):
            return pltpu.make_async_copy(
                support_hbm.at[src_r.at[j]], rows.at[j], gsem.at[j])

        def d_desc(b, j):
            return pltpu.make_async_copy(
                dst_hbm.at[pl.ds(base_m + b * BLK, BLK)], dst_r.at[j],
                dsem.at[j])

        def w_desc(b, j):
            return pltpu.make_async_copy(
                w_hbm.at[pl.ds(base_m + b * BLK, BLK)], w_r.at[j],
                wsem.at[j])

        def s_desc(b, j):
            return pltpu.make_async_copy(
                rows.at[j], acc.at[dst_r.at[j]], ssem.at[j])

        def mult(buf_ref, w_ref, nrow):
            def grp(t, c2):
                wv = w_ref[pl.ds(t * 16, 16)]
                for jj in range(16):
                    wj = wv[jj]
                    row = t * 16 + jj
                    for k in range(d // 16):
                        sl = pl.ds(k * 16, 16)
                        buf_ref[row, sl] = buf_ref[row, sl] * wj
                return c2

            lax.fori_loop(0, nrow // 16, grp, 0)

        for k in range(NBUF):
            x_desc(k, k).start()
        plsc.subcore_barrier()
        for b in range(2):
            x_desc(b, b).wait()
            d_desc(b, b).start()
            w_desc(b, b).start()
            g_desc(b, b).start()

        def step(b, j):
            g_desc(b, j).wait()
            d_desc(b, j).wait()
            w_desc(b, j).wait()

            @pl.when(b + NBUF < nmain)
            def _():
                x_desc(b + NBUF, j).start()

            jn = (j + 2) % NBUF

            @pl.when(b >= 2)
            def _():
                s_desc(b - 2, jn).wait()

            @pl.when(b + 2 < nmain)
            def _():
                x_desc(b + 2, jn).wait()
                d_desc(b + 2, jn).start()
                w_desc(b + 2, jn).start()
                g_desc(b + 2, jn).start()

            mult(rows.at[j], w_r.at[j], BLK)
            s_desc(b, j).start(add=True)

        nquad = nmain // NBUF

        def quad(q, c):
            for j in range(NBUF):
                step(q * NBUF + j, j)
            return c

        lax.fori_loop(0, nquad, quad, 0)
        for b in range(nquad * NBUF, nmain):
            step(b, b % NBUF)
        s_desc(nmain - 2, (nmain - 2) % NBUF).wait()
        s_desc(nmain - 1, (nmain - 1) % NBUF).wait()

        plsc.subcore_barrier()
        pltpu.sync_copy(
            acc.at[pl.ds(off, rows_pt)],
            out_hbm.at[cid, pl.ds(off, rows_pt)],
        )
        if rows_rem:
            @pl.when(sid == 0)
            def _():
                pltpu.sync_copy(
                    acc.at[pl.ds(rows_pt * NS, rows_rem)],
                    out_hbm.at[cid, pl.ds(rows_pt * NS, rows_rem)],
                )

    return seg


def _loss_body(p_ref, lbl_ref, w_ref, b_ref, out_ref):
    i = pl.program_id(0)
    ng = pl.num_programs(0)
    n_total = p_ref.shape[1] * ng
    emb = p_ref[0] + p_ref[1]
    logits = jnp.dot(emb, w_ref[...], preferred_element_type=jnp.float32)
    logits = logits + b_ref[...]
    m = jnp.max(logits, axis=1, keepdims=True)
    ex = jnp.exp(logits - m)
    lse = jnp.log(jnp.sum(ex, axis=1, keepdims=True)) + m
    col = lax.broadcasted_iota(jnp.int32, logits.shape, 1)
    oh = col == lbl_ref[...]
    part = jnp.sum(jnp.where(oh, logits, 0.0)) - jnp.sum(lse)

    @pl.when(i == 0)
    def _():
        out_ref[0, 0] = 0.0

    out_ref[0, 0] += part

    @pl.when(i == ng - 1)
    def _():
        out_ref[0, 0] = -out_ref[0, 0] / n_total


def _loss(partials, labels2d, w_disc, b2):
    _, n, d = partials.shape
    nparts = w_disc.shape[1]
    r = 2000
    grid = n // r
    assert grid * r == n
    return pl.pallas_call(
        _loss_body,
        grid=(grid,),
        in_specs=[
            pl.BlockSpec((NC, r, d), lambda i: (0, i, 0)),
            pl.BlockSpec((r, 1), lambda i: (i, 0)),
            pl.BlockSpec((d, nparts), lambda i: (0, 0)),
            pl.BlockSpec((1, nparts), lambda i: (0, 0)),
        ],
        out_specs=pl.BlockSpec((1, 1), lambda i: (0, 0),
                               memory_space=pltpu.SMEM),
        out_shape=jax.ShapeDtypeStruct((1, 1), jnp.float32),
    )(partials, labels2d, w_disc, b2)


def kernel(encoder_features, adj_weight, W_gcn, W_disc, b_disc, edge_index,
           pseudo_labels):
    n, d = encoder_features.shape
    e = edge_index.shape[1]

    support = _support_matmul(encoder_features, W_gcn)

    seg = _make_seg_sum(n, d, e)
    partials = seg(support, edge_index[0], edge_index[1], adj_weight)

    labels2d = pseudo_labels.astype(jnp.int32)[:, None]
    acc = _loss(partials, labels2d, W_disc, b_disc[None, :])
    return acc[0, 0]
